# Initial kernel scaffold; baseline (speedup 1.0000x reference)
import jax
import jax.numpy as jnp
from jax.experimental import pallas as pl


def kernel(x, edge_index, edge_attr, batch, We, be, Wee, bee, Wc, bc, W1, b1, gam, bet, W2, b2, Wg, bg, Wf, bf):
    raise NotImplementedError("write your pallas kernel here")


# speedup vs baseline: 2.1468x; 2.1468x over previous
import functools

import jax
import jax.numpy as jnp
from jax import lax
from jax.experimental import pallas as pl
from jax.experimental.pallas import tpu as pltpu
from jax.experimental.pallas import tpu_sc as plsc

N = 10000
E = 160000
D = 256
DH = 128
G = 64

NT = 16
EPT = E // NT
K = 80
NCH = EPT // K


def _edge_phase(hA, hB, eA, eB, src, dst, zrows):
    mesh = plsc.VectorSubcoreMesh(core_axis_name="c", subcore_axis_name="s")

    @functools.partial(
        pl.kernel,
        mesh=mesh,
        out_type=(
            jax.ShapeDtypeStruct((N, DH), jnp.float32),
            jax.ShapeDtypeStruct((N, DH), jnp.float32),
        ),
        scratch_types=[
            pltpu.VMEM((K,), jnp.int32),
            pltpu.VMEM((K,), jnp.int32),
            pltpu.VMEM((K, DH), jnp.float32),
            pltpu.VMEM((K, DH), jnp.float32),
            pltpu.VMEM_SHARED((N, DH), jnp.float32),
            pltpu.SemaphoreType.DMA,
        ],
    )
    def edge_k(hA_h, hB_h, eA_h, eB_h, src_h, dst_h, z_h, aA_h, aB_h,
               sidx, didx, gbuf, ebuf, accum, sem):
        c = lax.axis_index("c")
        s = lax.axis_index("s")

        def run(h_h, e_h, a_h):
            @pl.when(s == 0)
            def _():
                pltpu.sync_copy(z_h, accum)

            plsc.subcore_barrier()

            def chunk(k, carry):
                base = pl.multiple_of(s * EPT + k * K, 16)
                pltpu.sync_copy(src_h.at[pl.ds(base, K)], sidx)
                pltpu.sync_copy(dst_h.at[pl.ds(base, K)], didx)
                pltpu.async_copy(h_h.at[sidx], gbuf, sem).wait()
                pltpu.sync_copy(e_h.at[pl.ds(base, K)], ebuf)

                def row(r, cc):
                    for j in range(DH // 16):
                        sl = pl.ds(j * 16, 16)
                        v = gbuf[r, sl] + ebuf[r, sl]
                        gbuf[r, sl] = jnp.maximum(v, 0.0)
                    return cc

                lax.fori_loop(0, K, row, 0)
                pltpu.sync_copy(gbuf, accum.at[didx], add=True)
                return carry

            lax.fori_loop(0, NCH, chunk, 0)
            plsc.subcore_barrier()

            @pl.when(s == 0)
            def _():
                pltpu.sync_copy(accum, a_h)

        @pl.when(c == 0)
        def _():
            run(hA_h, eA_h, aA_h)

        @pl.when(c == 1)
        def _():
            run(hB_h, eB_h, aB_h)

    return edge_k(hA, hB, eA, eB, src, dst, zrows)


def _enc_body(x_ref, wA_ref, wB_ref, b_ref, hA_ref, hB_ref):
    x = x_ref[...]
    b = b_ref[...]
    hA_ref[...] = jnp.dot(x, wA_ref[...], preferred_element_type=jnp.float32) + b[:, :DH]
    hB_ref[...] = jnp.dot(x, wB_ref[...], preferred_element_type=jnp.float32) + b[:, DH:]


def _node_encode(x, We, be):
    R = 2000
    return pl.pallas_call(
        _enc_body,
        grid=(N // R,),
        in_specs=[
            pl.BlockSpec((R, D), lambda i: (i, 0)),
            pl.BlockSpec((D, DH), lambda i: (0, 0)),
            pl.BlockSpec((D, DH), lambda i: (0, 0)),
            pl.BlockSpec((1, D), lambda i: (0, 0)),
        ],
        out_specs=[
            pl.BlockSpec((R, DH), lambda i: (i, 0)),
            pl.BlockSpec((R, DH), lambda i: (i, 0)),
        ],
        out_shape=[
            jax.ShapeDtypeStruct((N, DH), jnp.float32),
            jax.ShapeDtypeStruct((N, DH), jnp.float32),
        ],
    )(x, We[:, :DH], We[:, DH:], be.reshape(1, D))


def _edge_encode(edge_attr, Wee, bee):
    R = 4000
    de = Wee.shape[0]
    return pl.pallas_call(
        _enc_body,
        grid=(E // R,),
        in_specs=[
            pl.BlockSpec((R, de), lambda i: (i, 0)),
            pl.BlockSpec((de, DH), lambda i: (0, 0)),
            pl.BlockSpec((de, DH), lambda i: (0, 0)),
            pl.BlockSpec((1, D), lambda i: (0, 0)),
        ],
        out_specs=[
            pl.BlockSpec((R, DH), lambda i: (i, 0)),
            pl.BlockSpec((R, DH), lambda i: (i, 0)),
        ],
        out_shape=[
            jax.ShapeDtypeStruct((E, DH), jnp.float32),
            jax.ShapeDtypeStruct((E, DH), jnp.float32),
        ],
    )(edge_attr, Wee[:, :DH], Wee[:, DH:], bee.reshape(1, D))


def _k1_body(hA_ref, hB_ref, aA_ref, aB_ref, wcA_ref, wcB_ref, bc_ref,
             w1_ref, b1_ref, z_ref, s_ref, q_ref):
    uA = hA_ref[...] + aA_ref[...]
    uB = hB_ref[...] + aB_ref[...]
    hc = (jnp.dot(uA, wcA_ref[...], preferred_element_type=jnp.float32)
          + jnp.dot(uB, wcB_ref[...], preferred_element_type=jnp.float32)
          + bc_ref[...])
    z = jnp.dot(hc, w1_ref[...], preferred_element_type=jnp.float32) + b1_ref[...]
    z_ref[...] = z

    @pl.when(pl.program_id(0) == 0)
    def _():
        s_ref[...] = jnp.zeros_like(s_ref)
        q_ref[...] = jnp.zeros_like(q_ref)

    s_ref[...] += jnp.sum(z, axis=0, keepdims=True)
    q_ref[...] += jnp.sum(z * z, axis=0, keepdims=True)


def _layer_k1(hA, hB, aA, aB, Wc, bc, W1, b1):
    R = 2000
    D2 = 2 * D
    return pl.pallas_call(
        _k1_body,
        grid=(N // R,),
        in_specs=[
            pl.BlockSpec((R, DH), lambda i: (i, 0)),
            pl.BlockSpec((R, DH), lambda i: (i, 0)),
            pl.BlockSpec((R, DH), lambda i: (i, 0)),
            pl.BlockSpec((R, DH), lambda i: (i, 0)),
            pl.BlockSpec((DH, D), lambda i: (0, 0)),
            pl.BlockSpec((DH, D), lambda i: (0, 0)),
            pl.BlockSpec((1, D), lambda i: (0, 0)),
            pl.BlockSpec((D, D2), lambda i: (0, 0)),
            pl.BlockSpec((1, D2), lambda i: (0, 0)),
        ],
        out_specs=[
            pl.BlockSpec((R, D2), lambda i: (i, 0)),
            pl.BlockSpec((1, D2), lambda i: (0, 0)),
            pl.BlockSpec((1, D2), lambda i: (0, 0)),
        ],
        out_shape=[
            jax.ShapeDtypeStruct((N, D2), jnp.float32),
            jax.ShapeDtypeStruct((1, D2), jnp.float32),
            jax.ShapeDtypeStruct((1, D2), jnp.float32),
        ],
    )(hA, hB, aA, aB, Wc[:DH, :], Wc[DH:, :], bc.reshape(1, D),
      W1, b1.reshape(1, D2))


def _k2_body(z_ref, s_ref, q_ref, gam_ref, bet_ref, w2A_ref, w2B_ref,
             b2_ref, hA_ref, hB_ref, oA_ref, oB_ref):
    mu = s_ref[...] / N
    var = q_ref[...] / N - mu * mu
    inv = lax.rsqrt(var + 1e-5)
    zn = (z_ref[...] - mu) * (inv * gam_ref[...]) + bet_ref[...]
    zl = jnp.where(zn >= 0, zn, 0.01 * zn)
    b2 = b2_ref[...]
    oA_ref[...] = (jnp.dot(zl, w2A_ref[...], preferred_element_type=jnp.float32)
                   + b2[:, :DH] + hA_ref[...])
    oB_ref[...] = (jnp.dot(zl, w2B_ref[...], preferred_element_type=jnp.float32)
                   + b2[:, DH:] + hB_ref[...])


def _layer_k2(z, ssum, ssq, gam, bet, W2, b2, hA, hB):
    R = 2000
    D2 = 2 * D
    return pl.pallas_call(
        _k2_body,
        grid=(N // R,),
        in_specs=[
            pl.BlockSpec((R, D2), lambda i: (i, 0)),
            pl.BlockSpec((1, D2), lambda i: (0, 0)),
            pl.BlockSpec((1, D2), lambda i: (0, 0)),
            pl.BlockSpec((1, D2), lambda i: (0, 0)),
            pl.BlockSpec((1, D2), lambda i: (0, 0)),
            pl.BlockSpec((D2, DH), lambda i: (0, 0)),
            pl.BlockSpec((D2, DH), lambda i: (0, 0)),
            pl.BlockSpec((1, D), lambda i: (0, 0)),
            pl.BlockSpec((R, DH), lambda i: (i, 0)),
            pl.BlockSpec((R, DH), lambda i: (i, 0)),
        ],
        out_specs=[
            pl.BlockSpec((R, DH), lambda i: (i, 0)),
            pl.BlockSpec((R, DH), lambda i: (i, 0)),
        ],
        out_shape=[
            jax.ShapeDtypeStruct((N, DH), jnp.float32),
            jax.ShapeDtypeStruct((N, DH), jnp.float32),
        ],
    )(z, ssum, ssq, gam.reshape(1, D2), bet.reshape(1, D2),
      W2[:, :DH], W2[:, DH:], b2.reshape(1, D), hA, hB)


def _pool_body(hA_ref, hB_ref, b_ref, wgA_ref, wgB_ref, bg_ref,
               wfA_ref, wfB_ref, bf_ref, o_ref):
    hA = hA_ref[...]
    hB = hB_ref[...]
    gate = (jnp.sum(hA * wgA_ref[...], axis=1, keepdims=True)
            + jnp.sum(hB * wgB_ref[...], axis=1, keepdims=True)
            + bg_ref[0, 0])
    seg = b_ref[...]
    gids = lax.broadcasted_iota(jnp.int32, (N, G), 1)
    M = seg == gids
    gm = jnp.max(jnp.where(M, gate, jnp.float32(-1e30)), axis=0, keepdims=True)
    gmax_n = jnp.sum(jnp.where(M, gm, 0.0), axis=1, keepdims=True)
    ex = jnp.exp(gate - gmax_n)
    den = jnp.sum(jnp.where(M, ex, 0.0), axis=0, keepdims=True)
    den_n = jnp.sum(jnp.where(M, den, 0.0), axis=1, keepdims=True)
    w = ex / den_n
    Mf = M.astype(jnp.float32)
    dn = (((0,), (0,)), ((), ()))
    pA = lax.dot_general(Mf, hA * w, dn, preferred_element_type=jnp.float32)
    pB = lax.dot_general(Mf, hB * w, dn, preferred_element_type=jnp.float32)
    t = (jnp.dot(pA, wfA_ref[...], preferred_element_type=jnp.float32)
         + jnp.dot(pB, wfB_ref[...], preferred_element_type=jnp.float32)
         + bf_ref[0, 0])
    o_ref[...] = 1.0 / (1.0 + jnp.exp(-t))


def _pool(hA, hB, batch32, Wg, bg, Wf, bf):
    return pl.pallas_call(
        _pool_body,
        out_shape=jax.ShapeDtypeStruct((G, 1), jnp.float32),
    )(hA, hB, batch32, Wg[:DH, 0].reshape(1, DH), Wg[DH:, 0].reshape(1, DH),
      bg.reshape(1, 1),
      Wf[:DH, :], Wf[DH:, :], bf.reshape(1, 1))


def kernel(x, edge_index, edge_attr, batch, We, be, Wee, bee, Wc, bc,
           W1, b1, gam, bet, W2, b2, Wg, bg, Wf, bf):
    src = edge_index[0].astype(jnp.int32)
    dst = edge_index[1].astype(jnp.int32)
    batch32 = batch.astype(jnp.int32).reshape(N, 1)
    zrows = jnp.zeros((N, DH), jnp.float32)

    hA, hB = _node_encode(x, We, be)
    eA, eB = _edge_encode(edge_attr, Wee, bee)

    for i in range(Wc.shape[0]):
        aA, aB = _edge_phase(hA, hB, eA, eB, src, dst, zrows)
        z, ssum, ssq = _layer_k1(hA, hB, aA, aB, Wc[i], bc[i], W1[i], b1[i])
        hA, hB = _layer_k2(z, ssum, ssq, gam[i], bet[i], W2[i], b2[i], hA, hB)

    return _pool(hA, hB, batch32, Wg, bg, Wf, bf)
